# baseline (speedup 1.0000x reference)
import functools

import jax
import jax.numpy as jnp
from jax import lax
from jax.experimental import pallas as pl
from jax.experimental.pallas import tpu as pltpu
from jax.experimental.pallas import tpu_sc as plsc

N = 10000
E = 160000
D = 256
DE = 16
G = 64

NC = 2
NS = 16
HALF = D // NC
EPT = E // NS
K = 80
NCH = EPT // K
WBT = 10
RPT = N // WBT


def _bf16_bits(x):
    b = lax.bitcast_convert_type(x, jnp.int32)
    lsb = lax.bitwise_and(lax.shift_right_logical(b, 16), 1)
    b = b + 0x7FFF + lsb
    return lax.bitwise_and(b, jnp.int32(-65536))


def _elin_body(ea_ref, w1_ref, w2_ref, b1_ref, b2_ref, out1_ref, out2_ref):
    c = pl.program_id(0)
    QU = HALF // 2
    ea = ea_ref[...].astype(jnp.bfloat16)

    def one(w_ref, b_ref, out_ref):
        ab = jnp.dot(ea, w_ref[...].astype(jnp.bfloat16),
                     preferred_element_type=jnp.float32) + b_ref[pl.ds(c, 1)]
        a = _bf16_bits(ab[:, :QU])
        b = _bf16_bits(ab[:, QU:])
        out_ref[0] = lax.bitwise_or(lax.shift_right_logical(a, 16), b)

    one(w1_ref, b1_ref, out1_ref)
    one(w2_ref, b2_ref, out2_ref)


def _elin2x(edge_attr, W1ab, W2ab, b1ab, b2ab):
    BE = 4000
    QU = HALF // 2
    out2 = [
        pl.BlockSpec((1, BE, QU), lambda c, i: (c, i, 0)),
        pl.BlockSpec((1, BE, QU), lambda c, i: (c, i, 0)),
    ]
    return pl.pallas_call(
        _elin_body,
        grid=(NC, E // BE),
        in_specs=[
            pl.BlockSpec((BE, DE), lambda c, i: (i, 0)),
            pl.BlockSpec((DE, HALF), lambda c, i: (c, 0)),
            pl.BlockSpec((DE, HALF), lambda c, i: (c, 0)),
            pl.BlockSpec((NC, HALF), lambda c, i: (0, 0)),
            pl.BlockSpec((NC, HALF), lambda c, i: (0, 0)),
        ],
        out_specs=out2,
        out_shape=[jax.ShapeDtypeStruct((NC, E, QU), jnp.int32)] * 2,
    )(edge_attr, W1ab, W2ab, b1ab, b2ab)


def _msg_body(xpk, elin2d, comb, zrows, out,
              cidx0, cidx1, gbuf0, gbuf1, ebuf0, ebuf1, sbuf0, sbuf1, agg,
              gsem0, gsem1, esem0, esem1, ssem0, ssem1, isem):
    c = lax.axis_index("c")
    s = lax.axis_index("s")

    @pl.when(s < WBT)
    def _():
        pltpu.sync_copy(zrows, agg.at[pl.ds(s * RPT, RPT)])

    plsc.subcore_barrier()

    ebase = c * E + s * EPT

    def elin_rows(j):
        return elin2d.at[pl.ds(ebase + j * K, K)]

    def compute(gbuf, ebuf, sbuf):
        mask = jnp.full((16,), -65536, jnp.int32)

        def row(r, _):
            for g in range(HALF // 32):
                gw = gbuf[r, pl.ds(16 * g, 16)]
                ew = ebuf[r, pl.ds(16 * g, 16)]
                fa = (plsc.bitcast(lax.shift_left(gw, 16), jnp.float32)
                      + plsc.bitcast(lax.shift_left(ew, 16), jnp.float32))
                fb = (plsc.bitcast(lax.bitwise_and(gw, mask), jnp.float32)
                      + plsc.bitcast(lax.bitwise_and(ew, mask), jnp.float32))
                sbuf[r, pl.ds(16 * g, 16)] = jnp.maximum(fa, 0.0)
                sbuf[r, pl.ds(64 + 16 * g, 16)] = jnp.maximum(fb, 0.0)
            return 0

        lax.fori_loop(0, K, row, 0)

    def step(j, cur, nxt):
        (cidx_c, gbuf_c, ebuf_c, sbuf_c, gsem_c, esem_c, ssem_c) = cur
        (cidx_n, gbuf_n, ebuf_n, sbuf_n, gsem_n, esem_n, ssem_n) = nxt

        @pl.when(j >= 1)
        def _():
            pltpu.make_async_copy(sbuf_n, agg.at[cidx_n.at[1]], ssem_n).wait()

        @pl.when(j + 1 < NCH)
        def _():
            pltpu.async_copy(comb.at[c, s, j + 1], cidx_n, isem)

        pltpu.make_async_copy(xpk.at[cidx_c.at[0]], gbuf_c, gsem_c).wait()
        pltpu.make_async_copy(elin_rows(j), ebuf_c, esem_c).wait()

        @pl.when(j + 1 < NCH)
        def _():
            pltpu.make_async_copy(comb.at[c, s, 0], cidx_n, isem).wait()
            pltpu.async_copy(xpk.at[cidx_n.at[0]], gbuf_n, gsem_n)
            pltpu.async_copy(elin_rows(j + 1), ebuf_n, esem_n)

        compute(gbuf_c, ebuf_c, sbuf_c)
        pltpu.async_copy(sbuf_c, agg.at[cidx_c.at[1]], ssem_c, add=True)

    slot0 = (cidx0, gbuf0, ebuf0, sbuf0, gsem0, esem0, ssem0)
    slot1 = (cidx1, gbuf1, ebuf1, sbuf1, gsem1, esem1, ssem1)

    pltpu.sync_copy(comb.at[c, s, 0], cidx0)
    pltpu.async_copy(xpk.at[cidx0.at[0]], gbuf0, gsem0)
    pltpu.async_copy(elin_rows(0), ebuf0, esem0)

    def pair(t, _):
        step(2 * t, slot0, slot1)
        step(2 * t + 1, slot1, slot0)
        return 0

    lax.fori_loop(0, NCH // 2, pair, 0)
    if NCH % 2:
        step(NCH - 1, slot0, slot1)
        pltpu.make_async_copy(sbuf0, agg.at[cidx0.at[1]], ssem0).wait()
    else:
        pltpu.make_async_copy(sbuf1, agg.at[cidx1.at[1]], ssem1).wait()
    plsc.subcore_barrier()

    @pl.when(s < WBT)
    def _():
        pltpu.sync_copy(agg.at[pl.ds(s * RPT, RPT)],
                        out.at[c, pl.ds(s * RPT, RPT)])


def _msg(xpk, elin2d, comb, zrows):
    mesh = plsc.VectorSubcoreMesh(core_axis_name="c", subcore_axis_name="s")
    kern = pl.kernel(
        _msg_body,
        mesh=mesh,
        out_type=jax.ShapeDtypeStruct((NC, N, HALF), jnp.float32),
        scratch_types=[
            pltpu.VMEM((2, K), jnp.int32),
            pltpu.VMEM((2, K), jnp.int32),
            pltpu.VMEM((K, HALF // 2), jnp.int32),
            pltpu.VMEM((K, HALF // 2), jnp.int32),
            pltpu.VMEM((K, HALF // 2), jnp.int32),
            pltpu.VMEM((K, HALF // 2), jnp.int32),
            pltpu.VMEM((K, HALF), jnp.float32),
            pltpu.VMEM((K, HALF), jnp.float32),
            pltpu.VMEM_SHARED((N, HALF), jnp.float32),
        ] + [pltpu.SemaphoreType.DMA] * 7,
        compiler_params=pltpu.CompilerParams(needs_layout_passes=False,
                                             use_tc_tiling_on_sc=False),
    )
    return kern(xpk, elin2d, comb, zrows)


def _pack_words(a, b):
    return lax.bitwise_or(lax.shift_right_logical(_bf16_bits(a), 16),
                          _bf16_bits(b))


def _stageb1_body(x_ref, agg_ref, wa, ba, wb, bb, wp, bp, gs, beta,
                  h_ref, hpk_ref):
    QU = HALF // 2
    agg = jnp.concatenate([agg_ref[0], agg_ref[1]], axis=1)
    hin = x_ref[...] + agg
    t = jnp.maximum(jnp.dot(hin, wa[...], preferred_element_type=jnp.float32)
                    + ba[...], 0.0)
    u = jnp.maximum(jnp.dot(t, wb[...], preferred_element_type=jnp.float32)
                    + bb[...], 0.0)
    v = u * gs[...] + beta[...]
    res = jnp.dot(x_ref[...], wp[...], preferred_element_type=jnp.float32) + bp[...]
    h = v + res
    h_ref[...] = h
    hpk_ref[0] = _pack_words(h[:, :QU], h[:, QU:HALF])
    hpk_ref[1] = _pack_words(h[:, HALF:HALF + QU], h[:, HALF + QU:])


def _stageb1(x, agg, Wa, ba, Wb, bb, Wp, bp, gs, beta):
    NB = 1000
    full = lambda i: (0, 0)
    return pl.pallas_call(
        _stageb1_body,
        grid=(N // NB,),
        in_specs=[
            pl.BlockSpec((NB, D), lambda i: (i, 0)),
            pl.BlockSpec((NC, NB, HALF), lambda i: (0, i, 0)),
            pl.BlockSpec((D, D), full),
            pl.BlockSpec((1, D), full),
            pl.BlockSpec((D, D), full),
            pl.BlockSpec((1, D), full),
            pl.BlockSpec((D, D), full),
            pl.BlockSpec((1, D), full),
            pl.BlockSpec((1, D), full),
            pl.BlockSpec((1, D), full),
        ],
        out_specs=[
            pl.BlockSpec((NB, D), lambda i: (i, 0)),
            pl.BlockSpec((NC, NB, HALF // 2), lambda i: (0, i, 0)),
        ],
        out_shape=[
            jax.ShapeDtypeStruct((N, D), jnp.float32),
            jax.ShapeDtypeStruct((NC, N, HALF // 2), jnp.int32),
        ],
    )(x, agg, Wa, ba, Wb, bb, Wp, bp, gs, beta)


def _pack_body(x_ref, out_ref):
    QU = HALF // 2
    h = x_ref[...]
    out_ref[0] = _pack_words(h[:, :QU], h[:, QU:HALF])
    out_ref[1] = _pack_words(h[:, HALF:HALF + QU], h[:, HALF + QU:])


def _pack(xR):
    NB = 1000
    return pl.pallas_call(
        _pack_body,
        grid=(N // NB,),
        in_specs=[pl.BlockSpec((NB, D), lambda i: (i, 0))],
        out_specs=pl.BlockSpec((NC, NB, HALF // 2), lambda i: (0, i, 0)),
        out_shape=jax.ShapeDtypeStruct((NC, N, HALF // 2), jnp.int32),
    )(xR)


def _stageb2_body(x_ref, agg_ref, wa, ba, wb, bb, wp, bp, gs, beta, batch_ref,
                  out_ref):
    i = pl.program_id(0)
    agg = jnp.concatenate([agg_ref[0], agg_ref[1]], axis=1)
    hin = x_ref[...] + agg
    t = jnp.maximum(jnp.dot(hin, wa[...], preferred_element_type=jnp.float32)
                    + ba[...], 0.0)
    u = jnp.maximum(jnp.dot(t, wb[...], preferred_element_type=jnp.float32)
                    + bb[...], 0.0)
    v = u * gs[...] + beta[...]
    res = jnp.dot(x_ref[...], wp[...], preferred_element_type=jnp.float32) + bp[...]
    h2 = v + res
    nb = h2.shape[0]
    bmat = jnp.broadcast_to(batch_ref[...], (nb, 128))
    gids = lax.broadcasted_iota(jnp.int32, (nb, 128), 1)
    onehot = jnp.where(bmat == gids, 1.0, 0.0).astype(jnp.float32)
    part = lax.dot_general(onehot, h2, (((0,), (0,)), ((), ())),
                           preferred_element_type=jnp.float32)

    @pl.when(i == 0)
    def _():
        out_ref[...] = part

    @pl.when(i != 0)
    def _():
        out_ref[...] = out_ref[...] + part


def _stageb2(x, agg, Wa, ba, Wb, bb, Wp, bp, gs, beta, batch2d):
    NB = 1000
    full = lambda i: (0, 0)
    return pl.pallas_call(
        _stageb2_body,
        grid=(N // NB,),
        in_specs=[
            pl.BlockSpec((NB, D), lambda i: (i, 0)),
            pl.BlockSpec((NC, NB, HALF), lambda i: (0, i, 0)),
            pl.BlockSpec((D, D), full),
            pl.BlockSpec((1, D), full),
            pl.BlockSpec((D, D), full),
            pl.BlockSpec((1, D), full),
            pl.BlockSpec((D, D), full),
            pl.BlockSpec((1, D), full),
            pl.BlockSpec((1, D), full),
            pl.BlockSpec((1, D), full),
            pl.BlockSpec((NB, 1), lambda i: (i, 0)),
        ],
        out_specs=pl.BlockSpec((128, D), full),
        out_shape=jax.ShapeDtypeStruct((128, D), jnp.float32),
    )(x, agg, Wa, ba, Wb, bb, Wp, bp, gs, beta, batch2d)


def kernel(x, edge_index, edge_attr, batch,
           W1a, b1a, W1b, b1b, We1, be1,
           W2a, b2a, W2b, b2b, We2, be2,
           Wp1, bp1, Wp2, bp2, g1, beta1, g2, beta2):
    src = edge_index[0]
    dst = edge_index[1]

    src3 = src.reshape(NS, NCH, K)
    dst3 = dst.reshape(NS, NCH, K)
    comb = jnp.stack([jnp.stack([src3, dst3], axis=2),
                      jnp.stack([src3 + N, dst3], axis=2)])
    zrows = jnp.zeros((RPT, HALF), jnp.float32)

    bn_scale = 1.0 / jnp.sqrt(1.0 + 1e-5)
    gs1 = (g1 * bn_scale).reshape(1, D)
    gs2 = (g2 * bn_scale).reshape(1, D)

    ii = jnp.arange(HALF // 2)
    a_off = 32 * (ii // 16) + ii % 16
    b_off = a_off + 16
    colsA = (jnp.arange(NC)[:, None] * HALF + a_off[None, :]).reshape(-1)
    colsB = (jnp.arange(NC)[:, None] * HALF + b_off[None, :]).reshape(-1)

    QU = HALF // 2

    def elin_parts(We, be):
        wa = We[:, colsA].reshape(DE, NC, QU)
        wb = We[:, colsB].reshape(DE, NC, QU)
        w = jnp.concatenate([wa, wb], axis=2)
        w = w.transpose(1, 0, 2).reshape(NC * DE, HALF)
        bv = jnp.concatenate([be[colsA].reshape(NC, QU),
                              be[colsB].reshape(NC, QU)], axis=1)
        return w, bv

    W1ab, b1ab = elin_parts(We1, be1)
    W2ab, b2ab = elin_parts(We2, be2)
    elin1, elin2 = _elin2x(edge_attr, W1ab, W2ab, b1ab, b2ab)
    elin1 = elin1.reshape(NC * E, QU)
    elin2 = elin2.reshape(NC * E, QU)

    cA = colsA.reshape(NC, QU)
    cB = colsB.reshape(NC, QU)
    R = jnp.concatenate([jnp.concatenate([cA[c], cB[c]]) for c in range(NC)])

    xR = x[:, R]
    xpk = _pack(xR).reshape(NC * N, QU)
    agg1 = _msg(xpk, elin1, comb, zrows)

    hR, hpk = _stageb1(xR, agg1, W1a[R, :], b1a.reshape(1, D),
                       W1b[:, R], b1b[R].reshape(1, D),
                       Wp1[R][:, R], bp1[R].reshape(1, D),
                       gs1[:, R], beta1[R].reshape(1, D))

    agg2 = _msg(hpk.reshape(NC * N, QU), elin2, comb, zrows)

    out128 = _stageb2(hR, agg2, W2a[R, :], b2a.reshape(1, D),
                      W2b, b2b.reshape(1, D),
                      Wp2[R, :], bp2.reshape(1, D),
                      gs2, beta2.reshape(1, D),
                      batch.reshape(N, 1))
    return out128[:G]


# speedup vs baseline: 1.7373x; 1.7373x over previous
import functools

import jax
import jax.numpy as jnp
from jax import lax
from jax.experimental import pallas as pl
from jax.experimental.pallas import tpu as pltpu
from jax.experimental.pallas import tpu_sc as plsc

N = 10000
E = 160000
D = 256
DE = 16
G = 64

NC = 2
NS = 16
HALF = D // NC
EPT = E // NS
K = 80
NCH = EPT // K
WBT = 10
RPT = N // WBT


def _bf16_bits(x):
    b = lax.bitcast_convert_type(x, jnp.int32)
    lsb = lax.bitwise_and(lax.shift_right_logical(b, 16), 1)
    b = b + 0x7FFF + lsb
    return lax.bitwise_and(b, jnp.int32(-65536))


def _elin_body(ea_ref, w1_ref, w2_ref, b1_ref, b2_ref, out1_ref, out2_ref):
    c = pl.program_id(0)
    QU = HALF // 2
    ea = ea_ref[...].astype(jnp.bfloat16)

    def one(w_ref, b_ref, out_ref):
        ab = jnp.dot(ea, w_ref[...].astype(jnp.bfloat16),
                     preferred_element_type=jnp.float32) + b_ref[pl.ds(c, 1)]
        a = _bf16_bits(ab[:, :QU])
        b = _bf16_bits(ab[:, QU:])
        out_ref[0] = lax.bitwise_or(lax.shift_right_logical(a, 16), b)

    one(w1_ref, b1_ref, out1_ref)
    one(w2_ref, b2_ref, out2_ref)


def _elin2x(edge_attr, W1ab, W2ab, b1ab, b2ab):
    BE = 4000
    QU = HALF // 2
    out2 = [
        pl.BlockSpec((1, BE, QU), lambda c, i: (c, i, 0)),
        pl.BlockSpec((1, BE, QU), lambda c, i: (c, i, 0)),
    ]
    return pl.pallas_call(
        _elin_body,
        grid=(NC, E // BE),
        in_specs=[
            pl.BlockSpec((BE, DE), lambda c, i: (i, 0)),
            pl.BlockSpec((DE, HALF), lambda c, i: (c, 0)),
            pl.BlockSpec((DE, HALF), lambda c, i: (c, 0)),
            pl.BlockSpec((NC, HALF), lambda c, i: (0, 0)),
            pl.BlockSpec((NC, HALF), lambda c, i: (0, 0)),
        ],
        out_specs=out2,
        out_shape=[jax.ShapeDtypeStruct((NC, E, QU), jnp.int32)] * 2,
    )(edge_attr, W1ab, W2ab, b1ab, b2ab)


def _msg_body(xcat, elin2d, comb, zrows, out,
              cidx0, cidx1, gbuf0, gbuf1, ebuf0, ebuf1, agg,
              gsem0, gsem1, esem0, esem1, ssem0, ssem1, isem):
    c = lax.axis_index("c")
    s = lax.axis_index("s")

    @pl.when(s < WBT)
    def _():
        pltpu.sync_copy(zrows, agg.at[pl.ds(s * RPT, RPT)])

    plsc.subcore_barrier()

    ebase = c * E + s * EPT

    def elin_rows(j):
        return elin2d.at[pl.ds(ebase + j * K, K)]

    def compute(gbuf, ebuf):
        mask = jnp.full((16,), -65536, jnp.int32)

        def row(r, _):
            for g in range(HALF // 32):
                w = ebuf[r, pl.ds(16 * g, 16)]
                ea = plsc.bitcast(lax.shift_left(w, 16), jnp.float32)
                eb = plsc.bitcast(lax.bitwise_and(w, mask), jnp.float32)
                sla = pl.ds(32 * g, 16)
                slb = pl.ds(32 * g + 16, 16)
                gbuf[r, sla] = jnp.maximum(gbuf[r, sla] + ea, 0.0)
                gbuf[r, slb] = jnp.maximum(gbuf[r, slb] + eb, 0.0)
            return 0

        lax.fori_loop(0, K, row, 0)

    def step(j, cur, nxt):
        (cidx_c, gbuf_c, ebuf_c, gsem_c, esem_c, ssem_c) = cur
        (cidx_n, gbuf_n, ebuf_n, gsem_n, esem_n, ssem_n) = nxt

        @pl.when(j >= 1)
        def _():
            pltpu.make_async_copy(gbuf_n, agg.at[cidx_n.at[1]], ssem_n).wait()

        @pl.when(j + 1 < NCH)
        def _():
            pltpu.async_copy(comb.at[c, s, j + 1], cidx_n, isem)

        pltpu.make_async_copy(xcat.at[cidx_c.at[0]], gbuf_c, gsem_c).wait()
        pltpu.make_async_copy(elin_rows(j), ebuf_c, esem_c).wait()

        @pl.when(j + 1 < NCH)
        def _():
            pltpu.make_async_copy(comb.at[c, s, 0], cidx_n, isem).wait()
            pltpu.async_copy(xcat.at[cidx_n.at[0]], gbuf_n, gsem_n)
            pltpu.async_copy(elin_rows(j + 1), ebuf_n, esem_n)

        compute(gbuf_c, ebuf_c)
        pltpu.async_copy(gbuf_c, agg.at[cidx_c.at[1]], ssem_c, add=True)

    slot0 = (cidx0, gbuf0, ebuf0, gsem0, esem0, ssem0)
    slot1 = (cidx1, gbuf1, ebuf1, gsem1, esem1, ssem1)

    pltpu.sync_copy(comb.at[c, s, 0], cidx0)
    pltpu.async_copy(xcat.at[cidx0.at[0]], gbuf0, gsem0)
    pltpu.async_copy(elin_rows(0), ebuf0, esem0)

    def pair(t, _):
        step(2 * t, slot0, slot1)
        step(2 * t + 1, slot1, slot0)
        return 0

    lax.fori_loop(0, NCH // 2, pair, 0)
    if NCH % 2:
        step(NCH - 1, slot0, slot1)
        pltpu.make_async_copy(gbuf0, agg.at[cidx0.at[1]], ssem0).wait()
    else:
        pltpu.make_async_copy(gbuf1, agg.at[cidx1.at[1]], ssem1).wait()
    plsc.subcore_barrier()

    @pl.when(s < WBT)
    def _():
        pltpu.sync_copy(agg.at[pl.ds(s * RPT, RPT)],
                        out.at[c, pl.ds(s * RPT, RPT)])


def _msg(xcat, elin2d, comb, zrows):
    mesh = plsc.VectorSubcoreMesh(core_axis_name="c", subcore_axis_name="s")
    kern = pl.kernel(
        _msg_body,
        mesh=mesh,
        out_type=jax.ShapeDtypeStruct((NC, N, HALF), jnp.float32),
        scratch_types=[
            pltpu.VMEM((2, K), jnp.int32),
            pltpu.VMEM((2, K), jnp.int32),
            pltpu.VMEM((K, HALF), jnp.float32),
            pltpu.VMEM((K, HALF), jnp.float32),
            pltpu.VMEM((K, HALF // 2), jnp.int32),
            pltpu.VMEM((K, HALF // 2), jnp.int32),
            pltpu.VMEM_SHARED((N, HALF), jnp.float32),
        ] + [pltpu.SemaphoreType.DMA] * 7,
        compiler_params=pltpu.CompilerParams(needs_layout_passes=False),
    )
    return kern(xcat, elin2d, comb, zrows)


def _pack_words(a, b):
    return lax.bitwise_or(lax.shift_right_logical(_bf16_bits(a), 16),
                          _bf16_bits(b))


def _stageb1_body(x_ref, agg_ref, wa, ba, wb, bb, wp, bp, gs, beta,
                  h_ref, hh_ref):
    agg = jnp.concatenate([agg_ref[0], agg_ref[1]], axis=1)
    hin = x_ref[...] + agg
    t = jnp.maximum(jnp.dot(hin, wa[...], preferred_element_type=jnp.float32)
                    + ba[...], 0.0)
    u = jnp.maximum(jnp.dot(t, wb[...], preferred_element_type=jnp.float32)
                    + bb[...], 0.0)
    v = u * gs[...] + beta[...]
    res = jnp.dot(x_ref[...], wp[...], preferred_element_type=jnp.float32) + bp[...]
    h = v + res
    h_ref[...] = h
    hh_ref[0] = h[:, :HALF]
    hh_ref[1] = h[:, HALF:]


def _stageb1(x, agg, Wa, ba, Wb, bb, Wp, bp, gs, beta):
    NB = 1000
    full = lambda i: (0, 0)
    return pl.pallas_call(
        _stageb1_body,
        grid=(N // NB,),
        in_specs=[
            pl.BlockSpec((NB, D), lambda i: (i, 0)),
            pl.BlockSpec((NC, NB, HALF), lambda i: (0, i, 0)),
            pl.BlockSpec((D, D), full),
            pl.BlockSpec((1, D), full),
            pl.BlockSpec((D, D), full),
            pl.BlockSpec((1, D), full),
            pl.BlockSpec((D, D), full),
            pl.BlockSpec((1, D), full),
            pl.BlockSpec((1, D), full),
            pl.BlockSpec((1, D), full),
        ],
        out_specs=[
            pl.BlockSpec((NB, D), lambda i: (i, 0)),
            pl.BlockSpec((NC, NB, HALF), lambda i: (0, i, 0)),
        ],
        out_shape=[
            jax.ShapeDtypeStruct((N, D), jnp.float32),
            jax.ShapeDtypeStruct((NC, N, HALF), jnp.float32),
        ],
    )(x, agg, Wa, ba, Wb, bb, Wp, bp, gs, beta)


def _stageb2_body(x_ref, agg_ref, wa, ba, wb, bb, wp, bp, gs, beta, batch_ref,
                  out_ref):
    i = pl.program_id(0)
    agg = jnp.concatenate([agg_ref[0], agg_ref[1]], axis=1)
    hin = x_ref[...] + agg
    t = jnp.maximum(jnp.dot(hin, wa[...], preferred_element_type=jnp.float32)
                    + ba[...], 0.0)
    u = jnp.maximum(jnp.dot(t, wb[...], preferred_element_type=jnp.float32)
                    + bb[...], 0.0)
    v = u * gs[...] + beta[...]
    res = jnp.dot(x_ref[...], wp[...], preferred_element_type=jnp.float32) + bp[...]
    h2 = v + res
    nb = h2.shape[0]
    bmat = jnp.broadcast_to(batch_ref[...], (nb, 128))
    gids = lax.broadcasted_iota(jnp.int32, (nb, 128), 1)
    onehot = jnp.where(bmat == gids, 1.0, 0.0).astype(jnp.float32)
    part = lax.dot_general(onehot, h2, (((0,), (0,)), ((), ())),
                           preferred_element_type=jnp.float32)

    @pl.when(i == 0)
    def _():
        out_ref[...] = part

    @pl.when(i != 0)
    def _():
        out_ref[...] = out_ref[...] + part


def _stageb2(x, agg, Wa, ba, Wb, bb, Wp, bp, gs, beta, batch2d):
    NB = 1000
    full = lambda i: (0, 0)
    return pl.pallas_call(
        _stageb2_body,
        grid=(N // NB,),
        in_specs=[
            pl.BlockSpec((NB, D), lambda i: (i, 0)),
            pl.BlockSpec((NC, NB, HALF), lambda i: (0, i, 0)),
            pl.BlockSpec((D, D), full),
            pl.BlockSpec((1, D), full),
            pl.BlockSpec((D, D), full),
            pl.BlockSpec((1, D), full),
            pl.BlockSpec((D, D), full),
            pl.BlockSpec((1, D), full),
            pl.BlockSpec((1, D), full),
            pl.BlockSpec((1, D), full),
            pl.BlockSpec((NB, 1), lambda i: (i, 0)),
        ],
        out_specs=pl.BlockSpec((128, D), full),
        out_shape=jax.ShapeDtypeStruct((128, D), jnp.float32),
    )(x, agg, Wa, ba, Wb, bb, Wp, bp, gs, beta, batch2d)


def kernel(x, edge_index, edge_attr, batch,
           W1a, b1a, W1b, b1b, We1, be1,
           W2a, b2a, W2b, b2b, We2, be2,
           Wp1, bp1, Wp2, bp2, g1, beta1, g2, beta2):
    src = edge_index[0]
    dst = edge_index[1]

    src3 = src.reshape(NS, NCH, K)
    dst3 = dst.reshape(NS, NCH, K)
    comb = jnp.stack([jnp.stack([src3, dst3], axis=2),
                      jnp.stack([src3 + N, dst3], axis=2)])
    zrows = jnp.zeros((RPT, HALF), jnp.float32)

    bn_scale = 1.0 / jnp.sqrt(1.0 + 1e-5)
    gs1 = (g1 * bn_scale).reshape(1, D)
    gs2 = (g2 * bn_scale).reshape(1, D)

    ii = jnp.arange(HALF // 2)
    a_off = 32 * (ii // 16) + ii % 16
    b_off = a_off + 16
    colsA = (jnp.arange(NC)[:, None] * HALF + a_off[None, :]).reshape(-1)
    colsB = (jnp.arange(NC)[:, None] * HALF + b_off[None, :]).reshape(-1)

    QU = HALF // 2

    def elin_parts(We, be):
        wa = We[:, colsA].reshape(DE, NC, QU)
        wb = We[:, colsB].reshape(DE, NC, QU)
        w = jnp.concatenate([wa, wb], axis=2)
        w = w.transpose(1, 0, 2).reshape(NC * DE, HALF)
        bv = jnp.concatenate([be[colsA].reshape(NC, QU),
                              be[colsB].reshape(NC, QU)], axis=1)
        return w, bv

    W1ab, b1ab = elin_parts(We1, be1)
    W2ab, b2ab = elin_parts(We2, be2)
    elin1, elin2 = _elin2x(edge_attr, W1ab, W2ab, b1ab, b2ab)
    elin1 = elin1.reshape(NC * E, QU)
    elin2 = elin2.reshape(NC * E, QU)

    xcat = jnp.concatenate([x[:, :HALF], x[:, HALF:]], axis=0)
    agg1 = _msg(xcat, elin1, comb, zrows)

    h, hh = _stageb1(x, agg1, W1a, b1a.reshape(1, D), W1b, b1b.reshape(1, D),
                     Wp1, bp1.reshape(1, D), gs1, beta1.reshape(1, D))

    agg2 = _msg(hh.reshape(NC * N, HALF), elin2, comb, zrows)

    out128 = _stageb2(h, agg2, W2a, b2a.reshape(1, D), W2b, b2b.reshape(1, D),
                      Wp2, bp2.reshape(1, D), gs2, beta2.reshape(1, D),
                      batch.reshape(N, 1))
    return out128[:G]
